# baseline (speedup 1.0000x reference)
import functools

import jax
import jax.numpy as jnp
from jax import lax
from jax.experimental import pallas as pl
from jax.experimental.pallas import tpu as pltpu
from jax.experimental.pallas import tpu_sc as plsc

_LMBDA = 0.0001
_LANES = 16
_HALF = 256


def _sc_scores(pos_h, pos_t, pos_r, neg_h, neg_t, neg_r, ent2, rel2):
  B = pos_h.shape[0]
  H = ent2.shape[2]
  info = plsc.get_sparse_core_info()
  nc, ns = info.num_cores, info.num_subcores
  nw = nc * ns
  bw = B // nw
  n_halves = bw // _HALF
  n_groups = _HALF // _LANES

  mesh = plsc.VectorSubcoreMesh(core_axis_name="c", subcore_axis_name="s")

  @functools.partial(
      pl.kernel,
      out_type=(
          jax.ShapeDtypeStruct((B,), jnp.float32),
          jax.ShapeDtypeStruct((B,), jnp.float32),
          jax.ShapeDtypeStruct((nw, _LANES), jnp.float32),
      ),
      mesh=mesh,
      compiler_params=pltpu.CompilerParams(
          use_tc_tiling_on_sc=False, needs_layout_passes=False),
      scratch_types=[
          pltpu.VMEM((bw,), jnp.int32),
          pltpu.VMEM((bw,), jnp.int32),
          pltpu.VMEM((bw,), jnp.int32),
          pltpu.VMEM((_HALF, 2, 64), jnp.float32),
          pltpu.VMEM((_HALF, 2, 64), jnp.float32),
          pltpu.VMEM((_HALF, 2, 64), jnp.float32),
          pltpu.VMEM((bw,), jnp.float32),
          pltpu.VMEM((_LANES,), jnp.float32),
          pltpu.SemaphoreType.DMA,
      ],
  )
  def k(ph, pt, pr, nh, nt, nr, ent_hbm, rel_hbm,
        ps_out, ns_out, reg_out,
        ih_v, it_v, ir_v, h_v, t_v, r_v, sc_v, acc_v, sem):
    wid = lax.axis_index("s") * nc + lax.axis_index("c")
    base = wid * bw
    lane = lax.iota(jnp.int32, _LANES)

    def phase(ih_hbm, it_hbm, ir_hbm, out_hbm, sq):
      pltpu.sync_copy(ih_hbm.at[pl.ds(base, bw)], ih_v)
      pltpu.sync_copy(it_hbm.at[pl.ds(base, bw)], it_v)
      pltpu.sync_copy(ir_hbm.at[pl.ds(base, bw)], ir_v)

      for half in range(n_halves):
        hb = half * _HALF
        copies = []
        for w in range(n_groups):
          sl = pl.ds(hb + w * _LANES, _LANES)
          dsl = pl.ds(w * _LANES, _LANES)
          copies.append(
              pltpu.async_copy(ent_hbm.at[ih_v[sl] >> 1], h_v.at[dsl], sem))
          copies.append(
              pltpu.async_copy(ent_hbm.at[it_v[sl] >> 1], t_v.at[dsl], sem))
          copies.append(
              pltpu.async_copy(rel_hbm.at[ir_v[sl] >> 1], r_v.at[dsl], sem))
        for cp in copies:
          cp.wait()

        def group(g, sq):
          gsl = pl.ds(hb + g * _LANES, _LANES)
          jvec = g * _LANES + lane
          par_h = ih_v[gsl] & 1
          par_t = it_v[gsl] & 1
          par_r = ir_v[gsl] & 1

          def kstep(kk, carry):
            score, sq = carry
            for u in range(4):
              ck = jnp.full((_LANES,), kk * 4 + u, jnp.int32)
              h = plsc.load_gather(h_v, [jvec, par_h, ck])
              t = plsc.load_gather(t_v, [jvec, par_t, ck])
              r = plsc.load_gather(r_v, [jvec, par_r, ck])
              score = score + h * r * t
              sq = sq + (h * h + t * t + r * r)
            return score, sq

          score, sq = lax.fori_loop(
              0, H // 4, kstep, (jnp.zeros((_LANES,), jnp.float32), sq))
          sc_v[gsl] = score
          return sq

        sq = lax.fori_loop(0, n_groups, group, sq)

      pltpu.sync_copy(sc_v, out_hbm.at[pl.ds(base, bw)])
      return sq

    sq = jnp.zeros((_LANES,), jnp.float32)
    sq = phase(ph, pt, pr, ps_out, sq)
    sq = phase(nh, nt, nr, ns_out, sq)
    acc_v[...] = sq
    pltpu.sync_copy(acc_v, reg_out.at[wid])

  return k(pos_h, pos_t, pos_r, neg_h, neg_t, neg_r, ent2, rel2)


def _loss_body(p_ref, n_ref, py_ref, ny_ref, reg_ref, out_ref, *, B, H):
  xp = -py_ref[...] * p_ref[...]
  xn = -ny_ref[...] * n_ref[...]
  sp = (jnp.maximum(xp, 0.0) + jnp.log(1.0 + jnp.exp(-jnp.abs(xp)))
        + jnp.maximum(xn, 0.0) + jnp.log(1.0 + jnp.exp(-jnp.abs(xn))))
  reg = jnp.sum(reg_ref[...])
  out_ref[0, 0] = jnp.sum(sp) * (1.0 / B) + _LMBDA * reg * (1.0 / (B * H))


def kernel(pos_h, pos_t, pos_r, neg_h, neg_t, neg_r, pos_y, neg_y,
           ent_embeddings, rel_embeddings):
  B = pos_h.shape[0]
  E, H = ent_embeddings.shape
  R = rel_embeddings.shape[0]
  ent2 = ent_embeddings.reshape(E // 2, 2, H)
  rel2 = rel_embeddings.reshape(R // 2, 2, H)
  p_score, n_score, reg = _sc_scores(
      pos_h, pos_t, pos_r, neg_h, neg_t, neg_r, ent2, rel2)
  rows = B // 128
  out = pl.pallas_call(
      functools.partial(_loss_body, B=B, H=H),
      out_shape=jax.ShapeDtypeStruct((1, 1), jnp.float32),
      out_specs=pl.BlockSpec(memory_space=pltpu.SMEM),
  )(p_score.reshape(rows, 128), n_score.reshape(rows, 128),
    pos_y.reshape(rows, 128), neg_y.reshape(rows, 128), reg)
  return out[0, 0]


# speedup vs baseline: 4.6371x; 4.6371x over previous
import functools

import jax
import jax.numpy as jnp
from jax import lax
from jax.experimental import pallas as pl
from jax.experimental.pallas import tpu as pltpu
from jax.experimental.pallas import tpu_sc as plsc

_LMBDA = 0.0001
_LANES = 16
_HALF = 256
_SUB = 8


def _sc_scores(pos_h, pos_t, pos_r, neg_h, neg_t, neg_r, ent3, rel3):
  B = pos_h.shape[0]
  H = ent3.shape[2]
  info = plsc.get_sparse_core_info()
  nc, ns = info.num_cores, info.num_subcores
  nw = nc * ns
  bw = B // nw
  n_halves = bw // _HALF
  n_groups = _HALF // _LANES
  n_hchunks = H // _LANES

  mesh = plsc.VectorSubcoreMesh(core_axis_name="c", subcore_axis_name="s")

  @functools.partial(
      pl.kernel,
      out_type=(
          jax.ShapeDtypeStruct((B,), jnp.float32),
          jax.ShapeDtypeStruct((B,), jnp.float32),
          jax.ShapeDtypeStruct((nw, _LANES), jnp.float32),
      ),
      mesh=mesh,
      compiler_params=pltpu.CompilerParams(needs_layout_passes=False),
      scratch_types=[
          pltpu.VMEM((_HALF,), jnp.int32),
          pltpu.VMEM((_HALF,), jnp.int32),
          pltpu.VMEM((_HALF,), jnp.int32),
          pltpu.VMEM((_HALF, 64), jnp.float32),
          pltpu.VMEM((_HALF, 64), jnp.float32),
          pltpu.VMEM((_HALF, 64), jnp.float32),
          pltpu.VMEM((bw,), jnp.float32),
          pltpu.VMEM((_LANES,), jnp.float32),
          pltpu.SemaphoreType.DMA,
      ],
  )
  def k(ph, pt, pr, nh, nt, nr, ent_hbm, rel_hbm,
        ps_out, ns_out, reg_out,
        ih_s, it_s, ir_s, h_v, t_v, r_v, sc_v, acc_v, sem):
    wid = lax.axis_index("s") * nc + lax.axis_index("c")
    base = wid * bw
    lane = lax.iota(jnp.int32, _LANES)

    def phase(ih_hbm, it_hbm, ir_hbm, out_hbm, sq):
      for half in range(n_halves):
        hb = half * _HALF
        pltpu.sync_copy(ih_hbm.at[pl.ds(base + hb, _HALF)], ih_s)
        pltpu.sync_copy(it_hbm.at[pl.ds(base + hb, _HALF)], it_s)
        pltpu.sync_copy(ir_hbm.at[pl.ds(base + hb, _HALF)], ir_s)

        def fire(i, carry):
          rb = i * _LANES
          ihvec = ih_s[pl.ds(rb, _LANES)]
          itvec = it_s[pl.ds(rb, _LANES)]
          irvec = ir_s[pl.ds(rb, _LANES)]
          copies = []
          for j in range(_LANES):
            row = rb + j
            eh = ihvec[j]
            et = itvec[j]
            er = irvec[j]
            copies.append(pltpu.async_copy(
                ent_hbm.at[eh >> 3, eh & 7], h_v.at[row], sem))
            copies.append(pltpu.async_copy(
                ent_hbm.at[et >> 3, et & 7], t_v.at[row], sem))
            copies.append(pltpu.async_copy(
                rel_hbm.at[er >> 3, er & 7], r_v.at[row], sem))
          for cp in copies:
            cp.wait()
          return carry

        lax.fori_loop(0, n_groups, fire, 0)

        def group(g, sq):
          score_vec = jnp.zeros((_LANES,), jnp.float32)
          for j in range(_LANES):
            row = g * _LANES + j
            s = None
            for c in range(n_hchunks):
              sl = pl.ds(c * _LANES, _LANES)
              h = h_v[row, sl]
              t = t_v[row, sl]
              r = r_v[row, sl]
              p = h * r * t
              s = p if s is None else s + p
              sq = sq + (h * h + t * t + r * r)
            for sh in (8, 4, 2, 1):
              s = s + jnp.take(s, lane ^ sh)
            score_vec = jnp.where(lane == j, s, score_vec)
          sc_v[pl.ds(hb + g * _LANES, _LANES)] = score_vec
          return sq

        sq = lax.fori_loop(0, n_groups, group, sq)

      pltpu.sync_copy(sc_v, out_hbm.at[pl.ds(base, bw)])
      return sq

    sq = jnp.zeros((_LANES,), jnp.float32)
    sq = phase(ph, pt, pr, ps_out, sq)
    sq = phase(nh, nt, nr, ns_out, sq)
    acc_v[...] = sq
    pltpu.sync_copy(acc_v, reg_out.at[wid])

  return k(pos_h, pos_t, pos_r, neg_h, neg_t, neg_r, ent3, rel3)


def _loss_body(p_ref, n_ref, py_ref, ny_ref, reg_ref, out_ref, *, B, H):
  xp = -py_ref[...] * p_ref[...]
  xn = -ny_ref[...] * n_ref[...]
  sp = (jnp.maximum(xp, 0.0) + jnp.log(1.0 + jnp.exp(-jnp.abs(xp)))
        + jnp.maximum(xn, 0.0) + jnp.log(1.0 + jnp.exp(-jnp.abs(xn))))
  reg = jnp.sum(reg_ref[...])
  out_ref[0, 0] = jnp.sum(sp) * (1.0 / B) + _LMBDA * reg * (1.0 / (B * H))


def kernel(pos_h, pos_t, pos_r, neg_h, neg_t, neg_r, pos_y, neg_y,
           ent_embeddings, rel_embeddings):
  B = pos_h.shape[0]
  E, H = ent_embeddings.shape
  R = rel_embeddings.shape[0]
  ent3 = ent_embeddings.reshape(E // _SUB, _SUB, H)
  rel3 = rel_embeddings.reshape(R // _SUB, _SUB, H)
  p_score, n_score, reg = _sc_scores(
      pos_h, pos_t, pos_r, neg_h, neg_t, neg_r, ent3, rel3)
  rows = B // 128
  out = pl.pallas_call(
      functools.partial(_loss_body, B=B, H=H),
      out_shape=jax.ShapeDtypeStruct((1, 1), jnp.float32),
      out_specs=pl.BlockSpec(memory_space=pltpu.SMEM),
  )(p_score.reshape(rows, 128), n_score.reshape(rows, 128),
    pos_y.reshape(rows, 128), neg_y.reshape(rows, 128), reg)
  return out[0, 0]


# speedup vs baseline: 5.0422x; 1.0874x over previous
import functools

import jax
import jax.numpy as jnp
from jax import lax
from jax.experimental import pallas as pl
from jax.experimental.pallas import tpu as pltpu
from jax.experimental.pallas import tpu_sc as plsc

_LMBDA = 0.0001
_LANES = 16
_CHUNK = 128
_SUB = 8


def _sc_scores(pos_h, pos_t, pos_r, neg_h, neg_t, neg_r, ent3, rel2):
  B = pos_h.shape[0]
  H = ent3.shape[2]
  RW = rel2.shape[0]
  info = plsc.get_sparse_core_info()
  nc, ns = info.num_cores, info.num_subcores
  nw = nc * ns
  bw = B // nw
  n_chunks = bw // _CHUNK
  n_groups = _CHUNK // _LANES
  n_hchunks = H // _LANES

  mesh = plsc.VectorSubcoreMesh(core_axis_name="c", subcore_axis_name="s")

  @functools.partial(
      pl.kernel,
      out_type=(
          jax.ShapeDtypeStruct((B,), jnp.float32),
          jax.ShapeDtypeStruct((B,), jnp.float32),
          jax.ShapeDtypeStruct((nw, _LANES), jnp.float32),
      ),
      mesh=mesh,
      compiler_params=pltpu.CompilerParams(needs_layout_passes=False),
      scratch_types=[
          pltpu.VMEM((_CHUNK,), jnp.int32),
          pltpu.VMEM((_CHUNK,), jnp.int32),
          pltpu.VMEM((_CHUNK,), jnp.int32),
          pltpu.VMEM((_CHUNK, 64), jnp.float32),
          pltpu.VMEM((_CHUNK, 64), jnp.float32),
          pltpu.VMEM((RW, 128), jnp.float32),
          pltpu.VMEM((bw,), jnp.float32),
          pltpu.VMEM((_LANES,), jnp.float32),
          pltpu.SemaphoreType.DMA,
      ],
  )
  def k(ph, pt, pr, nh, nt, nr, ent_hbm, rel_hbm,
        ps_out, ns_out, reg_out,
        ih_s, it_s, ir_s, h_v, t_v, rel_v, sc_v, acc_v, sem):
    wid = lax.axis_index("s") * nc + lax.axis_index("c")
    base = wid * bw
    lane = lax.iota(jnp.int32, _LANES)

    pltpu.sync_copy(rel_hbm, rel_v)

    def phase(ih_hbm, it_hbm, ir_hbm, out_hbm, sq):
      for chunk in range(n_chunks):
        cb = chunk * _CHUNK
        pltpu.sync_copy(ih_hbm.at[pl.ds(base + cb, _CHUNK)], ih_s)
        pltpu.sync_copy(it_hbm.at[pl.ds(base + cb, _CHUNK)], it_s)
        pltpu.sync_copy(ir_hbm.at[pl.ds(base + cb, _CHUNK)], ir_s)

        def fire(i, carry):
          rb = i * _LANES
          ihvec = ih_s[pl.ds(rb, _LANES)]
          itvec = it_s[pl.ds(rb, _LANES)]
          for j in range(_LANES):
            row = rb + j
            eh = ihvec[j]
            et = itvec[j]
            pltpu.async_copy(ent_hbm.at[eh >> 3, eh & 7], h_v.at[row], sem)
            pltpu.async_copy(ent_hbm.at[et >> 3, et & 7], t_v.at[row], sem)
          return carry

        lax.fori_loop(0, n_groups, fire, 0)

        def drain(i, carry):
          rb = i * _LANES
          for j in range(_LANES):
            row = rb + j
            pltpu.make_async_copy(ent_hbm.at[0, 0], h_v.at[row], sem).wait()
            pltpu.make_async_copy(ent_hbm.at[0, 0], t_v.at[row], sem).wait()
          return carry

        lax.fori_loop(0, n_groups, drain, 0)

        def group(g, sq):
          irvec = ir_s[pl.ds(g * _LANES, _LANES)]
          score_vec = jnp.zeros((_LANES,), jnp.float32)
          for j in range(_LANES):
            row = g * _LANES + j
            er = irvec[j]
            er2 = er >> 1
            eo = (er & 1) * 64
            s = None
            for c in range(n_hchunks):
              sl = pl.ds(c * _LANES, _LANES)
              h = h_v[row, sl]
              t = t_v[row, sl]
              r = rel_v[er2, pl.ds(eo + c * _LANES, _LANES)]
              p = h * r * t
              s = p if s is None else s + p
              sq = sq + (h * h + t * t + r * r)
            for sh in (8, 4, 2, 1):
              s = s + jnp.take(s, lane ^ sh)
            score_vec = jnp.where(lane == j, s, score_vec)
          sc_v[pl.ds(cb + g * _LANES, _LANES)] = score_vec
          return sq

        sq = lax.fori_loop(0, n_groups, group, sq)

      pltpu.sync_copy(sc_v, out_hbm.at[pl.ds(base, bw)])
      return sq

    sq = jnp.zeros((_LANES,), jnp.float32)
    sq = phase(ph, pt, pr, ps_out, sq)
    sq = phase(nh, nt, nr, ns_out, sq)
    acc_v[...] = sq
    pltpu.sync_copy(acc_v, reg_out.at[wid])

  return k(pos_h, pos_t, pos_r, neg_h, neg_t, neg_r, ent3, rel2)


def _loss_body(p_ref, n_ref, py_ref, ny_ref, reg_ref, out_ref, *, B, H):
  xp = -py_ref[...] * p_ref[...]
  xn = -ny_ref[...] * n_ref[...]
  sp = (jnp.maximum(xp, 0.0) + jnp.log(1.0 + jnp.exp(-jnp.abs(xp)))
        + jnp.maximum(xn, 0.0) + jnp.log(1.0 + jnp.exp(-jnp.abs(xn))))
  reg = jnp.sum(reg_ref[...])
  out_ref[0, 0] = jnp.sum(sp) * (1.0 / B) + _LMBDA * reg * (1.0 / (B * H))


def kernel(pos_h, pos_t, pos_r, neg_h, neg_t, neg_r, pos_y, neg_y,
           ent_embeddings, rel_embeddings):
  B = pos_h.shape[0]
  E, H = ent_embeddings.shape
  R = rel_embeddings.shape[0]
  ent3 = ent_embeddings.reshape(E // _SUB, _SUB, H)
  rel2 = rel_embeddings.reshape(R // 2, 2 * H)
  p_score, n_score, reg = _sc_scores(
      pos_h, pos_t, pos_r, neg_h, neg_t, neg_r, ent3, rel2)
  rows = B // 128
  out = pl.pallas_call(
      functools.partial(_loss_body, B=B, H=H),
      out_shape=jax.ShapeDtypeStruct((1, 1), jnp.float32),
      out_specs=pl.BlockSpec(memory_space=pltpu.SMEM),
  )(p_score.reshape(rows, 128), n_score.reshape(rows, 128),
    pos_y.reshape(rows, 128), neg_y.reshape(rows, 128), reg)
  return out[0, 0]
